# Initial kernel scaffold; baseline (speedup 1.0000x reference)
import jax
import jax.numpy as jnp
from jax.experimental import pallas as pl


def kernel(activations, labels, epoch, max_epochs, som_vectors, cell_labels, cell_reliability):
    raise NotImplementedError("write your pallas kernel here")


# speedup vs baseline: 1.3665x; 1.3665x over previous
import functools

import jax
import jax.numpy as jnp
from jax.experimental import pallas as pl
from jax.experimental.pallas import tpu as pltpu

INPUT_DIM = 64
GRID_N = 32
CELLS = GRID_N * GRID_N
BATCH = 4096
TILE = 512
NTILES = BATCH // TILE
SOM_LR_MAX = 0.2
SOM_LR_MIN = 0.05
CTX_MAX = 2
CTX_MIN = 0
REL_THRESH = 0.95


def _som_body(ctx_ref, lr_ref, x_ref, lab_ref, w_ref, clab_ref, crel_ref,
              err_ref, som_out_ref, avg_ref, s_ref, ssum_ref, msum_ref):
    i = pl.program_id(0)
    x = x_ref[...]
    w = w_ref[...]

    xn = jnp.sum(x * x, axis=1, keepdims=True)
    xw = jax.lax.dot_general(x, w, (((1,), (1,)), ((), ())),
                             preferred_element_type=jnp.float32)
    wn_rows = jax.lax.dot_general(
        jnp.ones((8, INPUT_DIM), jnp.float32), w * w, (((1,), (1,)), ((), ())),
        precision=jax.lax.Precision.HIGHEST,
        preferred_element_type=jnp.float32)
    d = jnp.maximum((xn + wn_rows[0:1]) - 2.0 * xw, 0.0)

    col = jax.lax.broadcasted_iota(jnp.int32, (TILE, CELLS), 1)
    min_d = jnp.min(d, axis=1, keepdims=True)
    bmu = jnp.min(jnp.where(d == min_d, col, CELLS), axis=1, keepdims=True)

    labm = clab_ref[...] != lab_ref[...]
    dc = jnp.where(labm, jnp.inf, d)
    min_dc = jnp.min(dc, axis=1, keepdims=True)
    cbmu = jnp.min(jnp.where(dc == min_dc, col, CELLS), axis=1, keepdims=True)

    ohc = (col == cbmu).astype(jnp.float32)
    cvec = jax.lax.dot_general(ohc, w, (((1,), (0,)), ((), ())),
                               precision=jax.lax.Precision.HIGHEST,
                               preferred_element_type=jnp.float32)
    rel = jnp.sum(jnp.where(col == cbmu, crel_ref[...], 0.0), axis=1,
                  keepdims=True) * 0.01
    valid = (rel >= REL_THRESH).astype(jnp.float32)
    err_ref[...] = 0.01 * rel * (cvec - x) * valid

    ohb = (col == bmu).astype(jnp.float32)
    x_aug = jnp.concatenate([x, jnp.ones((TILE, INPUT_DIM), jnp.float32)], axis=1)
    sp = jax.lax.dot_general(ohb, x_aug, (((0,), (0,)), ((), ())),
                             precision=jax.lax.Precision.HIGHEST,
                             preferred_element_type=jnp.float32)

    @pl.when(i == 0)
    def _init():
        ssum_ref[0] = 0.0
        msum_ref[0] = 0.0
        s_ref[...] = jnp.zeros_like(s_ref)

    s_ref[...] += sp
    ssum_ref[0] += jnp.sum(jnp.sqrt(min_d))
    msum_ref[0] += jnp.sum(min_d)

    @pl.when(i == NTILES - 1)
    def _finalize():
        ctx = ctx_ref[0]
        lr = lr_ref[0]
        r_i = jax.lax.broadcasted_iota(jnp.int32, (CELLS, CELLS), 0)
        c_i = jax.lax.broadcasted_iota(jnp.int32, (CELLS, CELLS), 1)
        dxm = jnp.abs((r_i & (GRID_N - 1)) - (c_i >> 5))
        dym = jnp.abs((r_i >> 5) - (c_i & (GRID_N - 1)))
        ch = jnp.maximum(dxm, dym)
        wgt = jnp.where(ch == 0, 1.0,
                        jnp.where(ch == 1, 0.5,
                                  jnp.where(ch == 2, 0.25, 0.0)))
        stencil = lr * wgt * (ch <= ctx).astype(jnp.float32)
        conv = jax.lax.dot_general(stencil, s_ref[...], (((1,), (0,)), ((), ())),
                                   precision=jax.lax.Precision.HIGHEST,
                                   preferred_element_type=jnp.float32)
        numer = conv[:, :INPUT_DIM]
        denom = conv[:, INPUT_DIM:INPUT_DIM + 1]
        upd = w + numer - w * denom
        gate = (msum_ref[0] * (1.0 / BATCH)) > 0.0001
        som_out_ref[...] = jnp.where(gate, upd, w)
        avg_ref[...] = jnp.full((1, 1), ssum_ref[0] * (1.0 / BATCH))


@functools.partial(jax.jit, static_argnames=())
def _som_step(activations, labels2d, ctx, lr, flat_som, clab_row, crel_row):
    kern = pl.pallas_call(
        _som_body,
        grid=(NTILES,),
        in_specs=[
            pl.BlockSpec(memory_space=pltpu.SMEM),
            pl.BlockSpec(memory_space=pltpu.SMEM),
            pl.BlockSpec((TILE, INPUT_DIM), lambda i: (i, 0)),
            pl.BlockSpec((TILE, 1), lambda i: (i, 0)),
            pl.BlockSpec((CELLS, INPUT_DIM), lambda i: (0, 0)),
            pl.BlockSpec((1, CELLS), lambda i: (0, 0)),
            pl.BlockSpec((1, CELLS), lambda i: (0, 0)),
        ],
        out_specs=[
            pl.BlockSpec((TILE, INPUT_DIM), lambda i: (i, 0)),
            pl.BlockSpec((CELLS, INPUT_DIM), lambda i: (0, 0)),
            pl.BlockSpec((1, 1), lambda i: (0, 0)),
        ],
        out_shape=[
            jax.ShapeDtypeStruct((BATCH, INPUT_DIM), jnp.float32),
            jax.ShapeDtypeStruct((CELLS, INPUT_DIM), jnp.float32),
            jax.ShapeDtypeStruct((1, 1), jnp.float32),
        ],
        scratch_shapes=[
            pltpu.VMEM((CELLS, 2 * INPUT_DIM), jnp.float32),
            pltpu.SMEM((1,), jnp.float32),
            pltpu.SMEM((1,), jnp.float32),
        ],
        compiler_params=pltpu.CompilerParams(
            dimension_semantics=("arbitrary",),
        ),
    )
    return kern(ctx, lr, activations, labels2d, flat_som, clab_row, crel_row)


def kernel(activations, labels, epoch, max_epochs, som_vectors, cell_labels,
           cell_reliability):
    epoch = jnp.asarray(epoch, jnp.float32)
    max_epochs = jnp.asarray(max_epochs, jnp.float32)
    progress = jnp.where(max_epochs > 0.0, (max_epochs - epoch) / max_epochs, 0.0)
    progress = jnp.clip(progress, 0.0, 1.0)
    som_context = CTX_MIN + (progress ** 4 * (CTX_MAX - CTX_MIN)).astype(jnp.int32)
    som_lr = SOM_LR_MIN + progress * (SOM_LR_MAX - SOM_LR_MIN)

    flat_som = som_vectors.reshape(CELLS, INPUT_DIM)
    labels2d = labels.astype(jnp.int32).reshape(BATCH, 1)
    clab_row = cell_labels.astype(jnp.int32).reshape(1, CELLS)
    crel_row = cell_reliability.reshape(1, CELLS)
    ctx = som_context.reshape(1)
    lr = som_lr.astype(jnp.float32).reshape(1)

    som_errors, new_flat, avg = _som_step(
        activations, labels2d, ctx, lr, flat_som, clab_row, crel_row)
    return (som_errors,
            new_flat.reshape(GRID_N, GRID_N, INPUT_DIM),
            avg.reshape(()))
